# Initial kernel scaffold; baseline (speedup 1.0000x reference)
import jax
import jax.numpy as jnp
from jax.experimental import pallas as pl


def kernel(z_e, embedding):
    raise NotImplementedError("write your pallas kernel here")


# speedup vs baseline: 1.3027x; 1.3027x over previous
import functools

import jax
import jax.numpy as jnp
from jax import lax
from jax.experimental import pallas as pl
from jax.experimental.pallas import tpu as pltpu
from jax.experimental.pallas import tpu_sc as plsc

_NUM_E = 8192
_DIM = 256
_B, _H, _W = 16, 32, 32
_ROWS = _B * _H * _W
_BETA = 0.25

_BR = 256
_G = _ROWS // _BR

_NC, _NS = 2, 16
_NW = _NC * _NS
_ROWS_PER_W = _ROWS // _NW
_CHUNK = 128

_DN_T = (((1,), (1,)), ((), ()))


def _argmin_body(z_ref, e_ref, zsq_ref, esq_ref, idx_ref, loss_ref):
    z = z_ref[...]
    mm = lax.dot_general(2.0 * z, e_ref[...], _DN_T,
                         preferred_element_type=jnp.float32)
    dist = (zsq_ref[...] - mm) + esq_ref[...]
    m = jnp.min(dist, axis=1, keepdims=True)
    ii = lax.broadcasted_iota(jnp.int32, dist.shape, 1)
    idx_ref[0, 0, :] = jnp.min(jnp.where(dist <= m, ii, _NUM_E), axis=1)

    @pl.when(pl.program_id(0) == 0)
    def _init():
        loss_ref[0, 0] = 0.0

    loss_ref[0, 0] += jnp.sum(m)


def _distance_argmin(z_flat, emb, z_sq, e_sq):
    return pl.pallas_call(
        _argmin_body,
        grid=(_G,),
        in_specs=[
            pl.BlockSpec((_BR, _DIM), lambda i: (i, 0)),
            pl.BlockSpec((_NUM_E, _DIM), lambda i: (0, 0)),
            pl.BlockSpec((_BR, 1), lambda i: (i, 0)),
            pl.BlockSpec((1, _NUM_E), lambda i: (0, 0)),
        ],
        out_specs=[
            pl.BlockSpec((1, 1, _BR), lambda i: (i, 0, 0)),
            pl.BlockSpec((1, 1), lambda i: (0, 0), memory_space=pltpu.SMEM),
        ],
        out_shape=[
            jax.ShapeDtypeStruct((_G, 1, _BR), jnp.int32),
            jax.ShapeDtypeStruct((1, 1), jnp.float32),
        ],
    )(z_flat, emb, z_sq, e_sq)


def _gather_body(emb_hbm, idx_hbm, out_hbm, idx_v, rows_v, sem):
    wid = lax.axis_index("s") * _NC + lax.axis_index("c")
    for j in range(_ROWS_PER_W // _CHUNK):
        base = wid * _ROWS_PER_W + j * _CHUNK
        pltpu.sync_copy(idx_hbm.at[pl.ds(base, _CHUNK)], idx_v)
        pltpu.async_copy(emb_hbm.at[idx_v], rows_v, sem).wait()
        pltpu.sync_copy(rows_v, out_hbm.at[pl.ds(base, _CHUNK)])


def _make_gather():
    return functools.partial(
        pl.kernel,
        mesh=plsc.VectorSubcoreMesh(core_axis_name="c", subcore_axis_name="s"),
        out_type=jax.ShapeDtypeStruct((_ROWS, _DIM), jnp.float32),
        scratch_types=[
            pltpu.VMEM((_CHUNK,), jnp.int32),
            pltpu.VMEM((_CHUNK, _DIM), jnp.float32),
            pltpu.SemaphoreType.DMA,
        ],
    )(_gather_body)


def kernel(z_e, embedding):
    z_flat = jnp.transpose(z_e, (0, 2, 3, 1)).reshape(_ROWS, _DIM)
    z_sq = jnp.sum(z_flat * z_flat, axis=1, keepdims=True)
    e_sq = jnp.sum(embedding * embedding, axis=1)[None, :]
    idx_blocks, loss_acc = _distance_argmin(z_flat, embedding, z_sq, e_sq)
    indices = idx_blocks.reshape(_ROWS)
    z_q_flat = _make_gather()(embedding, indices)
    z_q = z_q_flat.reshape(_B, _H, _W, _DIM).transpose(0, 3, 1, 2)
    loss = loss_acc[0, 0] * ((1.0 + _BETA) / (_ROWS * _DIM))
    return z_q, loss, indices.reshape(_B, _H, _W)
